# initial kernel scaffold (unmeasured)
import jax
import jax.numpy as jnp
from jax import lax
from jax.experimental import pallas as pl
from jax.experimental.pallas import tpu as pltpu

NZ = 4
CAP = 192


def _ag_kernel(x, router_t):
    t_loc, d = x.shape
    e_sh, d_r = router_t.shape

    def body(x_ref, r_ref, xall_ref, rall_ref, sx, rx, sr, rr):
        my_x = lax.axis_index("x")
        my_y = lax.axis_index("y")
        my_z = lax.axis_index("z")

        xall_ref[my_z] = x_ref[...]
        rall_ref[my_z] = r_ref[...]

        sends = []
        for h in range(1, NZ):
            dz = lax.rem(my_z + h, NZ)
            sd_x = pltpu.make_async_remote_copy(
                src_ref=x_ref,
                dst_ref=xall_ref.at[my_z],
                send_sem=sx.at[h - 1],
                recv_sem=rx.at[my_z],
                device_id=(my_x, my_y, dz),
                device_id_type=pl.DeviceIdType.MESH,
            )
            sd_r = pltpu.make_async_remote_copy(
                src_ref=r_ref,
                dst_ref=rall_ref.at[my_z],
                send_sem=sr.at[h - 1],
                recv_sem=rr.at[my_z],
                device_id=(my_x, my_y, dz),
                device_id_type=pl.DeviceIdType.MESH,
            )
            sd_x.start()
            sd_r.start()
            sends.append((sd_x, sd_r))

        for h in range(1, NZ):
            sz = lax.rem(my_z + (NZ - h), NZ)
            rc_x = pltpu.make_async_remote_copy(
                src_ref=x_ref,
                dst_ref=xall_ref.at[sz],
                send_sem=sx.at[h - 1],
                recv_sem=rx.at[sz],
                device_id=(my_x, my_y, sz),
                device_id_type=pl.DeviceIdType.MESH,
            )
            rc_r = pltpu.make_async_remote_copy(
                src_ref=r_ref,
                dst_ref=rall_ref.at[sz],
                send_sem=sr.at[h - 1],
                recv_sem=rr.at[sz],
                device_id=(my_x, my_y, sz),
                device_id_type=pl.DeviceIdType.MESH,
            )
            rc_x.wait_recv()
            rc_r.wait_recv()

        for sd_x, sd_r in sends:
            sd_x.wait_send()
            sd_r.wait_send()

    return pl.pallas_call(
        body,
        out_shape=[
            jax.ShapeDtypeStruct((NZ, t_loc, d), x.dtype),
            jax.ShapeDtypeStruct((NZ, e_sh, d_r), router_t.dtype),
        ],
        in_specs=[
            pl.BlockSpec(memory_space=pltpu.VMEM),
            pl.BlockSpec(memory_space=pltpu.VMEM),
        ],
        out_specs=[
            pl.BlockSpec(memory_space=pltpu.VMEM),
            pl.BlockSpec(memory_space=pltpu.VMEM),
        ],
        scratch_shapes=[
            pltpu.SemaphoreType.DMA((NZ,)),
            pltpu.SemaphoreType.DMA((NZ,)),
            pltpu.SemaphoreType.DMA((NZ,)),
            pltpu.SemaphoreType.DMA((NZ,)),
        ],
        compiler_params=pltpu.CompilerParams(collective_id=0),
    )(x, router_t)


def _ffn_kernel(onehot_d, x_all, w1, w2):
    e_loc, cap, t = onehot_d.shape
    _, d, f = w1.shape

    def body(oh_ref, xall_ref, w1_ref, w2_ref, y_ref):
        xd = jnp.dot(oh_ref[0], xall_ref[...], preferred_element_type=jnp.float32)
        h = jnp.maximum(
            jnp.dot(xd, w1_ref[0], preferred_element_type=jnp.float32), 0.0
        )
        y_ref[0] = jnp.dot(h, w2_ref[0], preferred_element_type=jnp.float32)

    return pl.pallas_call(
        body,
        grid=(e_loc,),
        out_shape=jax.ShapeDtypeStruct((e_loc, cap, d), jnp.float32),
        in_specs=[
            pl.BlockSpec((1, cap, t), lambda e: (e, 0, 0)),
            pl.BlockSpec((t, d), lambda e: (0, 0)),
            pl.BlockSpec((1, d, f), lambda e: (e, 0, 0)),
            pl.BlockSpec((1, f, d), lambda e: (e, 0, 0)),
        ],
        out_specs=pl.BlockSpec((1, cap, d), lambda e: (e, 0, 0)),
    )(onehot_d, x_all, w1, w2)


def _combine_rs_kernel(cmb, y_flat, t_loc):
    t, ec = cmb.shape
    _, d = y_flat.shape

    def body(c_ref, y_ref, out_ref, part_ref, recv_ref, ss, rs):
        my_x = lax.axis_index("x")
        my_y = lax.axis_index("y")
        my_z = lax.axis_index("z")

        partial = jnp.dot(c_ref[...], y_ref[...], preferred_element_type=jnp.float32)
        part_ref[...] = partial.reshape(NZ, t_loc, d)

        sends = []
        for h in range(1, NZ):
            dz = lax.rem(my_z + h, NZ)
            sd = pltpu.make_async_remote_copy(
                src_ref=part_ref.at[dz],
                dst_ref=recv_ref.at[my_z],
                send_sem=ss.at[h - 1],
                recv_sem=rs.at[my_z],
                device_id=(my_x, my_y, dz),
                device_id_type=pl.DeviceIdType.MESH,
            )
            sd.start()
            sends.append(sd)

        recv_ref[my_z] = part_ref[my_z]
        for h in range(1, NZ):
            sz = lax.rem(my_z + (NZ - h), NZ)
            rc = pltpu.make_async_remote_copy(
                src_ref=part_ref.at[my_z],
                dst_ref=recv_ref.at[sz],
                send_sem=ss.at[h - 1],
                recv_sem=rs.at[sz],
                device_id=(my_x, my_y, sz),
                device_id_type=pl.DeviceIdType.MESH,
            )
            rc.wait_recv()

        out_ref[...] = jnp.sum(recv_ref[...], axis=0)

        for sd in sends:
            sd.wait_send()

    return pl.pallas_call(
        body,
        out_shape=jax.ShapeDtypeStruct((t_loc, d), jnp.float32),
        in_specs=[
            pl.BlockSpec(memory_space=pltpu.VMEM),
            pl.BlockSpec(memory_space=pltpu.VMEM),
        ],
        out_specs=pl.BlockSpec(memory_space=pltpu.VMEM),
        scratch_shapes=[
            pltpu.VMEM((NZ, t_loc, d), jnp.float32),
            pltpu.VMEM((NZ, t_loc, d), jnp.float32),
            pltpu.SemaphoreType.DMA((NZ,)),
            pltpu.SemaphoreType.DMA((NZ,)),
        ],
        compiler_params=pltpu.CompilerParams(collective_id=1),
    )(cmb, y_flat)


def kernel(x, router, W1, W2):
    t_loc, d = x.shape
    e_loc = W1.shape[0]
    t = t_loc * NZ
    e = e_loc * NZ

    my_z = lax.axis_index("z")

    x_all4, r_all4 = _ag_kernel(x, router.T)
    x_all = x_all4.reshape(t, d)
    router_all_t = r_all4.reshape(e, d)

    gates = jnp.einsum("td,ed->te", x_all, router_all_t)
    g1 = jnp.max(gates, axis=1)
    top1 = jnp.argmax(gates, axis=1)
    masked = jnp.where(
        jax.nn.one_hot(top1, e, dtype=jnp.bool_), -jnp.inf, gates
    )
    g2 = jnp.max(masked, axis=1)
    top2 = jnp.argmax(masked, axis=1)
    w1g = 1.0 / (1.0 + jnp.exp(g2 - g1))
    w2g = 1.0 - w1g

    eids = my_z * e_loc + jnp.arange(e_loc)
    m1 = top1[:, None] == eids[None, :]
    m2 = top2[:, None] == eids[None, :]
    amask = m1 | m2
    wmask = m1 * w1g[:, None] + m2 * w2g[:, None]
    pos = jnp.cumsum(amask, axis=0) - 1
    pos = jnp.where(amask, pos, -1)

    caps = jnp.arange(CAP)
    onehot_d = (pos.T[:, None, :] == caps[None, :, None]).astype(jnp.float32)
    cmb = (pos[:, :, None] == caps[None, None, :]) * wmask[:, :, None]
    cmb = cmb.reshape(t, e_loc * CAP).astype(jnp.float32)

    y = _ffn_kernel(onehot_d, x_all, W1, W2)

    return _combine_rs_kernel(cmb, y.reshape(e_loc * CAP, d), t_loc)


# baseline (device time: 156072 ns/iter reference)
import jax
import jax.numpy as jnp
from jax import lax
from jax.experimental import pallas as pl
from jax.experimental.pallas import tpu as pltpu

NZ = 4
CAP = 192


def _ag_kernel(x, router_t):
    t_loc, d = x.shape
    e_sh, d_r = router_t.shape

    def body(x_ref, r_ref, xall_ref, rall_ref, sx, rx, sr, rr):
        my_x = lax.axis_index("x")
        my_y = lax.axis_index("y")
        my_z = lax.axis_index("z")

        xall_ref[my_z] = x_ref[...]
        rall_ref[my_z] = r_ref[...]

        sends = []
        for h in range(1, NZ):
            dz = lax.rem(my_z + h, NZ)
            sd_x = pltpu.make_async_remote_copy(
                src_ref=x_ref,
                dst_ref=xall_ref.at[my_z],
                send_sem=sx.at[h - 1],
                recv_sem=rx.at[my_z],
                device_id=(my_x, my_y, dz),
                device_id_type=pl.DeviceIdType.MESH,
            )
            sd_r = pltpu.make_async_remote_copy(
                src_ref=r_ref,
                dst_ref=rall_ref.at[my_z],
                send_sem=sr.at[h - 1],
                recv_sem=rr.at[my_z],
                device_id=(my_x, my_y, dz),
                device_id_type=pl.DeviceIdType.MESH,
            )
            sd_x.start()
            sd_r.start()
            sends.append((sd_x, sd_r))

        for h in range(1, NZ):
            sz = lax.rem(my_z + (NZ - h), NZ)
            rc_x = pltpu.make_async_remote_copy(
                src_ref=x_ref,
                dst_ref=xall_ref.at[sz],
                send_sem=sx.at[h - 1],
                recv_sem=rx.at[sz],
                device_id=(my_x, my_y, sz),
                device_id_type=pl.DeviceIdType.MESH,
            )
            rc_r = pltpu.make_async_remote_copy(
                src_ref=r_ref,
                dst_ref=rall_ref.at[sz],
                send_sem=sr.at[h - 1],
                recv_sem=rr.at[sz],
                device_id=(my_x, my_y, sz),
                device_id_type=pl.DeviceIdType.MESH,
            )
            rc_x.wait_recv()
            rc_r.wait_recv()

        for sd_x, sd_r in sends:
            sd_x.wait_send()
            sd_r.wait_send()

    return pl.pallas_call(
        body,
        out_shape=[
            jax.ShapeDtypeStruct((NZ, t_loc, d), x.dtype),
            jax.ShapeDtypeStruct((NZ, e_sh, d_r), router_t.dtype),
        ],
        in_specs=[
            pl.BlockSpec(memory_space=pltpu.VMEM),
            pl.BlockSpec(memory_space=pltpu.VMEM),
        ],
        out_specs=[
            pl.BlockSpec(memory_space=pltpu.VMEM),
            pl.BlockSpec(memory_space=pltpu.VMEM),
        ],
        scratch_shapes=[
            pltpu.SemaphoreType.DMA((NZ,)),
            pltpu.SemaphoreType.DMA((NZ,)),
            pltpu.SemaphoreType.DMA((NZ,)),
            pltpu.SemaphoreType.DMA((NZ,)),
        ],
    )(x, router_t)


def _ffn_kernel(onehot_d, x_all, w1, w2):
    e_loc, cap, t = onehot_d.shape
    _, d, f = w1.shape

    def body(oh_ref, xall_ref, w1_ref, w2_ref, y_ref):
        xd = jnp.dot(oh_ref[0], xall_ref[...], preferred_element_type=jnp.float32)
        h = jnp.maximum(
            jnp.dot(xd, w1_ref[0], preferred_element_type=jnp.float32), 0.0
        )
        y_ref[0] = jnp.dot(h, w2_ref[0], preferred_element_type=jnp.float32)

    return pl.pallas_call(
        body,
        grid=(e_loc,),
        out_shape=jax.ShapeDtypeStruct((e_loc, cap, d), jnp.float32),
        in_specs=[
            pl.BlockSpec((1, cap, t), lambda e: (e, 0, 0)),
            pl.BlockSpec((t, d), lambda e: (0, 0)),
            pl.BlockSpec((1, d, f), lambda e: (e, 0, 0)),
            pl.BlockSpec((1, f, d), lambda e: (e, 0, 0)),
        ],
        out_specs=pl.BlockSpec((1, cap, d), lambda e: (e, 0, 0)),
        compiler_params=pltpu.CompilerParams(
            vmem_limit_bytes=100 * 1024 * 1024
        ),
    )(onehot_d, x_all, w1, w2)


def _combine_rs_kernel(cmb, y_flat, t_loc):
    t, ec = cmb.shape
    _, d = y_flat.shape

    def body(c_ref, y_ref, out_ref, part_ref, recv_ref, ss, rs):
        my_x = lax.axis_index("x")
        my_y = lax.axis_index("y")
        my_z = lax.axis_index("z")

        partial = jnp.dot(c_ref[...], y_ref[...], preferred_element_type=jnp.float32)
        part_ref[...] = partial.reshape(NZ, t_loc, d)

        sends = []
        for h in range(1, NZ):
            dz = lax.rem(my_z + h, NZ)
            sd = pltpu.make_async_remote_copy(
                src_ref=part_ref.at[dz],
                dst_ref=recv_ref.at[my_z],
                send_sem=ss.at[h - 1],
                recv_sem=rs.at[my_z],
                device_id=(my_x, my_y, dz),
                device_id_type=pl.DeviceIdType.MESH,
            )
            sd.start()
            sends.append(sd)

        recv_ref[my_z] = part_ref[my_z]
        for h in range(1, NZ):
            sz = lax.rem(my_z + (NZ - h), NZ)
            rc = pltpu.make_async_remote_copy(
                src_ref=part_ref.at[my_z],
                dst_ref=recv_ref.at[sz],
                send_sem=ss.at[h - 1],
                recv_sem=rs.at[sz],
                device_id=(my_x, my_y, sz),
                device_id_type=pl.DeviceIdType.MESH,
            )
            rc.wait_recv()

        out_ref[...] = jnp.sum(recv_ref[...], axis=0)

        for sd in sends:
            sd.wait_send()

    return pl.pallas_call(
        body,
        out_shape=jax.ShapeDtypeStruct((t_loc, d), jnp.float32),
        in_specs=[
            pl.BlockSpec(memory_space=pltpu.VMEM),
            pl.BlockSpec(memory_space=pltpu.VMEM),
        ],
        out_specs=pl.BlockSpec(memory_space=pltpu.VMEM),
        scratch_shapes=[
            pltpu.VMEM((NZ, t_loc, d), jnp.float32),
            pltpu.VMEM((NZ, t_loc, d), jnp.float32),
            pltpu.SemaphoreType.DMA((NZ,)),
            pltpu.SemaphoreType.DMA((NZ,)),
        ],
    )(cmb, y_flat)


def kernel(x, router, W1, W2):
    t_loc, d = x.shape
    e_loc = W1.shape[0]
    t = t_loc * NZ
    e = e_loc * NZ

    my_z = lax.axis_index("z")

    x_all4, r_all4 = _ag_kernel(x, router.T)
    x_all = x_all4.reshape(t, d)
    router_all_t = r_all4.reshape(e, d)

    gates = jnp.einsum(
        "td,ed->te", x_all, router_all_t, precision=lax.Precision.HIGHEST
    )
    g1 = jnp.max(gates, axis=1)
    top1 = jnp.argmax(gates, axis=1)
    masked = jnp.where(
        jax.nn.one_hot(top1, e, dtype=jnp.bool_), -jnp.inf, gates
    )
    g2 = jnp.max(masked, axis=1)
    top2 = jnp.argmax(masked, axis=1)
    w1g = 1.0 / (1.0 + jnp.exp(g2 - g1))
    w2g = 1.0 - w1g

    eids = my_z * e_loc + jnp.arange(e_loc)
    m1 = top1[:, None] == eids[None, :]
    m2 = top2[:, None] == eids[None, :]
    amask = m1 | m2
    wmask = m1 * w1g[:, None] + m2 * w2g[:, None]
    pos = jnp.cumsum(amask, axis=0) - 1
    pos = jnp.where(amask, pos, -1)

    caps = jnp.arange(CAP)
    onehot_d = (pos.T[:, None, :] == caps[None, :, None]).astype(jnp.float32)
    cmb = (pos[:, :, None] == caps[None, None, :]) * wmask[:, :, None]
    cmb = cmb.reshape(t, e_loc * CAP).astype(jnp.float32)

    y = _ffn_kernel(onehot_d, x_all, W1, W2)

    return _combine_rs_kernel(cmb, y.reshape(e_loc * CAP, d), t_loc)
